# baseline (speedup 1.0000x reference)
import jax
import jax.numpy as jnp
from jax.experimental import pallas as pl
from jax.experimental.pallas import tpu as pltpu

_BETA = 0.25
_BT = 1024


def _vq_block(z_ref, e_ref, zq_ref, loss_ref, acc_ref,
              esq_ref, ehi_ref, emid_ref):
    i = pl.program_id(0)
    nsteps = pl.num_programs(0)
    zb = z_ref[...]
    e = e_ref[...]
    n = e.shape[0]

    @pl.when(i == 0)
    def _prep():
        esq_ref[...] = jnp.sum(e * e, axis=1, keepdims=True)
        e_hi = e.astype(jnp.bfloat16)
        ehi_ref[...] = e_hi
        emid_ref[...] = (e - e_hi.astype(jnp.float32)).astype(jnp.bfloat16)

    zsq = jnp.sum(zb * zb, axis=1, keepdims=True)
    esq = esq_ref[...]
    mm = jax.lax.dot_general(zb, e, (((1,), (1,)), ((), ())),
                             preferred_element_type=jnp.float32)
    dist = (zsq + esq[:, 0][None, :]) - 2.0 * mm
    dmin = jnp.min(dist, axis=1, keepdims=True)
    iota = jax.lax.broadcasted_iota(jnp.int32, dist.shape, 1)
    idx = jnp.min(jnp.where(dist == dmin, iota, jnp.int32(n)),
                  axis=1, keepdims=True)
    onehot = (iota == idx).astype(jnp.bfloat16)
    zq = (jax.lax.dot_general(onehot, ehi_ref[...], (((1,), (0,)), ((), ())),
                              preferred_element_type=jnp.float32)
          + jax.lax.dot_general(onehot, emid_ref[...], (((1,), (0,)), ((), ())),
                                preferred_element_type=jnp.float32))
    zq_ref[...] = zq

    part = jnp.sum(dmin)

    @pl.when(i == 0)
    def _init():
        acc_ref[0, 0] = part

    @pl.when(i > 0)
    def _acc():
        acc_ref[0, 0] += part

    @pl.when(i == nsteps - 1)
    def _fin():
        total = acc_ref[0, 0]
        denom = zb.shape[0] * zb.shape[1] * nsteps
        mean_sq = total / denom
        loss_ref[...] = jnp.full((1, 128), (1.0 + _BETA) * mean_sq,
                                 jnp.float32)


def kernel(z, emb_weight):
    B, C, H, W = z.shape
    N, D = emb_weight.shape
    zp = jnp.transpose(z, (0, 2, 3, 1))
    z_flat = zp.reshape(-1, D)
    T = z_flat.shape[0]
    nblk = T // _BT

    zq_flat, loss_out = pl.pallas_call(
        _vq_block,
        grid=(nblk,),
        in_specs=[
            pl.BlockSpec((_BT, D), lambda i: (i, 0)),
            pl.BlockSpec((N, D), lambda i: (0, 0)),
        ],
        out_specs=[
            pl.BlockSpec((_BT, D), lambda i: (i, 0)),
            pl.BlockSpec((1, 128), lambda i: (0, 0)),
        ],
        out_shape=[
            jax.ShapeDtypeStruct((T, D), jnp.float32),
            jax.ShapeDtypeStruct((1, 128), jnp.float32),
        ],
        scratch_shapes=[
            pltpu.SMEM((1, 1), jnp.float32),
            pltpu.VMEM((N, 1), jnp.float32),
            pltpu.VMEM((N, D), jnp.bfloat16),
            pltpu.VMEM((N, D), jnp.bfloat16),
        ],
    )(z_flat, emb_weight)

    z_quantise = jnp.transpose(zq_flat.reshape(zp.shape), (0, 3, 1, 2))
    return (z_quantise, loss_out[0, 0])


# speedup vs baseline: 1.0823x; 1.0823x over previous
import jax
import jax.numpy as jnp
from jax.experimental import pallas as pl
from jax.experimental.pallas import tpu as pltpu

_BETA = 0.25
_BT = 2048


def _vq_block(z_ref, e_ref, zq_ref, loss_ref, acc_ref):
    i = pl.program_id(0)
    nsteps = pl.num_programs(0)
    zb = z_ref[...]
    e = e_ref[...]
    n = e.shape[0]
    zsq = jnp.sum(zb * zb, axis=1, keepdims=True)
    esq = jnp.sum(e * e, axis=1)
    mm = jax.lax.dot_general(zb, e, (((1,), (1,)), ((), ())),
                             preferred_element_type=jnp.float32)
    dist = (zsq + esq[None, :]) - 2.0 * mm
    dmin = jnp.min(dist, axis=1, keepdims=True)
    iota = jax.lax.broadcasted_iota(jnp.int32, dist.shape, 1)
    idx = jnp.min(jnp.where(dist == dmin, iota, jnp.int32(n)),
                  axis=1, keepdims=True)
    onehot = (iota == idx).astype(jnp.bfloat16)
    e_hi = e.astype(jnp.bfloat16)
    e_mid = (e - e_hi.astype(jnp.float32)).astype(jnp.bfloat16)
    zq = (jax.lax.dot_general(onehot, e_hi, (((1,), (0,)), ((), ())),
                              preferred_element_type=jnp.float32)
          + jax.lax.dot_general(onehot, e_mid, (((1,), (0,)), ((), ())),
                                preferred_element_type=jnp.float32))
    zq_ref[...] = zq

    part = jnp.sum(dmin)

    @pl.when(i == 0)
    def _init():
        acc_ref[0, 0] = part

    @pl.when(i > 0)
    def _acc():
        acc_ref[0, 0] += part

    @pl.when(i == nsteps - 1)
    def _fin():
        total = acc_ref[0, 0]
        denom = zb.shape[0] * zb.shape[1] * nsteps
        mean_sq = total / denom
        loss_ref[...] = jnp.full((1, 128), (1.0 + _BETA) * mean_sq,
                                 jnp.float32)


def kernel(z, emb_weight):
    B, C, H, W = z.shape
    N, D = emb_weight.shape
    zp = jnp.transpose(z, (0, 2, 3, 1))
    z_flat = zp.reshape(-1, D)
    T = z_flat.shape[0]
    nblk = T // _BT

    zq_flat, loss_out = pl.pallas_call(
        _vq_block,
        grid=(nblk,),
        in_specs=[
            pl.BlockSpec((_BT, D), lambda i: (i, 0)),
            pl.BlockSpec((N, D), lambda i: (0, 0)),
        ],
        out_specs=[
            pl.BlockSpec((_BT, D), lambda i: (i, 0)),
            pl.BlockSpec((1, 128), lambda i: (0, 0)),
        ],
        out_shape=[
            jax.ShapeDtypeStruct((T, D), jnp.float32),
            jax.ShapeDtypeStruct((1, 128), jnp.float32),
        ],
        scratch_shapes=[pltpu.SMEM((1, 1), jnp.float32)],
    )(z_flat, emb_weight)

    z_quantise = jnp.transpose(zq_flat.reshape(zp.shape), (0, 3, 1, 2))
    return (z_quantise, loss_out[0, 0])


# speedup vs baseline: 1.0835x; 1.0011x over previous
import jax
import jax.numpy as jnp
from jax.experimental import pallas as pl
from jax.experimental.pallas import tpu as pltpu

_BETA = 0.25
_BT = 4096


def _vq_block(z_ref, e_ref, zq_ref, loss_ref, acc_ref):
    i = pl.program_id(0)
    nsteps = pl.num_programs(0)
    zb = z_ref[...]
    e = e_ref[...]
    n = e.shape[0]
    zsq = jnp.sum(zb * zb, axis=1, keepdims=True)
    esq = jnp.sum(e * e, axis=1)
    mm = jax.lax.dot_general(zb, e, (((1,), (1,)), ((), ())),
                             preferred_element_type=jnp.float32)
    dist = (zsq + esq[None, :]) - 2.0 * mm
    dmin = jnp.min(dist, axis=1, keepdims=True)
    iota = jax.lax.broadcasted_iota(jnp.int32, dist.shape, 1)
    idx = jnp.min(jnp.where(dist == dmin, iota, jnp.int32(n)),
                  axis=1, keepdims=True)
    onehot = (iota == idx).astype(jnp.bfloat16)
    e_hi = e.astype(jnp.bfloat16)
    e_mid = (e - e_hi.astype(jnp.float32)).astype(jnp.bfloat16)
    zq = (jax.lax.dot_general(onehot, e_hi, (((1,), (0,)), ((), ())),
                              preferred_element_type=jnp.float32)
          + jax.lax.dot_general(onehot, e_mid, (((1,), (0,)), ((), ())),
                                preferred_element_type=jnp.float32))
    zq_ref[...] = zq

    part = jnp.sum(dmin)

    @pl.when(i == 0)
    def _init():
        acc_ref[0, 0] = part

    @pl.when(i > 0)
    def _acc():
        acc_ref[0, 0] += part

    @pl.when(i == nsteps - 1)
    def _fin():
        total = acc_ref[0, 0]
        denom = zb.shape[0] * zb.shape[1] * nsteps
        mean_sq = total / denom
        loss_ref[...] = jnp.full((1, 128), (1.0 + _BETA) * mean_sq,
                                 jnp.float32)


def kernel(z, emb_weight):
    B, C, H, W = z.shape
    N, D = emb_weight.shape
    zp = jnp.transpose(z, (0, 2, 3, 1))
    z_flat = zp.reshape(-1, D)
    T = z_flat.shape[0]
    nblk = T // _BT

    zq_flat, loss_out = pl.pallas_call(
        _vq_block,
        grid=(nblk,),
        in_specs=[
            pl.BlockSpec((_BT, D), lambda i: (i, 0)),
            pl.BlockSpec((N, D), lambda i: (0, 0)),
        ],
        out_specs=[
            pl.BlockSpec((_BT, D), lambda i: (i, 0)),
            pl.BlockSpec((1, 128), lambda i: (0, 0)),
        ],
        out_shape=[
            jax.ShapeDtypeStruct((T, D), jnp.float32),
            jax.ShapeDtypeStruct((1, 128), jnp.float32),
        ],
        scratch_shapes=[pltpu.SMEM((1, 1), jnp.float32)],
    )(z_flat, emb_weight)

    z_quantise = jnp.transpose(zq_flat.reshape(zp.shape), (0, 3, 1, 2))
    return (z_quantise, loss_out[0, 0])
